# baseline (speedup 1.0000x reference)
import jax
import jax.numpy as jnp
from jax import lax
from jax.experimental import pallas as pl
from jax.experimental.pallas import tpu as pltpu
from jax.experimental.pallas import tpu_sc as plsc

_INFO = plsc.get_sparse_core_info()
_NC = _INFO.num_cores
_NS = _INFO.num_subcores
_NW = _NC * _NS

_TOTAL = 16 * 3 * 640 * 640
_PER_W = _TOTAL // _NW
_NBUF = 4
_N_CHUNKS = 20
_CHUNK = _PER_W // _N_CHUNKS


def _vsc_body(x_hbm, o_hbm, b0, b1, b2, b3, sin, sout):
    wid = lax.axis_index("s") * _NC + lax.axis_index("c")
    base = wid * _PER_W
    bufs = (b0, b1, b2, b3)
    ins = [
        pltpu.make_async_copy(
            x_hbm.at[pl.ds(base + j * _CHUNK, _CHUNK)],
            bufs[j % _NBUF],
            sin.at[j % _NBUF],
        )
        for j in range(_N_CHUNKS)
    ]
    outs = [
        pltpu.make_async_copy(
            bufs[j % _NBUF],
            o_hbm.at[pl.ds(base + j * _CHUNK, _CHUNK)],
            sout.at[j % _NBUF],
        )
        for j in range(_N_CHUNKS)
    ]
    for b in range(_NBUF - 1):
        ins[b].start()
    for j in range(_N_CHUNKS):
        if j + _NBUF - 1 < _N_CHUNKS:
            if j - 1 >= 0:
                outs[j - 1].wait()
            ins[j + _NBUF - 1].start()
        ins[j].wait()
        outs[j].start()
    for k in range(_N_CHUNKS - _NBUF, _N_CHUNKS):
        outs[k].wait()


def kernel(images):
    b, c, h, w = images.shape
    flat = images.reshape(_TOTAL)
    mesh = plsc.VectorSubcoreMesh(core_axis_name="c", subcore_axis_name="s")
    out = pl.kernel(
        _vsc_body,
        out_type=jax.ShapeDtypeStruct((_TOTAL,), jnp.float32),
        mesh=mesh,
        scratch_types=[
            pltpu.VMEM((_CHUNK,), jnp.float32),
            pltpu.VMEM((_CHUNK,), jnp.float32),
            pltpu.VMEM((_CHUNK,), jnp.float32),
            pltpu.VMEM((_CHUNK,), jnp.float32),
            pltpu.SemaphoreType.DMA((_NBUF,)),
            pltpu.SemaphoreType.DMA((_NBUF,)),
        ],
    )(flat)
    return out.reshape(b, c, h, w)


# speedup vs baseline: 1.0092x; 1.0092x over previous
import jax
import jax.numpy as jnp
from jax import lax
from jax.experimental import pallas as pl
from jax.experimental.pallas import tpu as pltpu
from jax.experimental.pallas import tpu_sc as plsc

_INFO = plsc.get_sparse_core_info()
_NC = _INFO.num_cores
_NS = _INFO.num_subcores
_NW = _NC * _NS

_TOTAL = 16 * 3 * 640 * 640
_PER_W = _TOTAL // _NW
_N_CHUNKS = 10
_CHUNK = _PER_W // _N_CHUNKS


def _vsc_body(x_hbm, o_hbm, buf0, buf1, sin, sout):
    wid = lax.axis_index("s") * _NC + lax.axis_index("c")
    base = wid * _PER_W
    bufs = (buf0, buf1)
    ins = [
        pltpu.make_async_copy(
            x_hbm.at[pl.ds(base + j * _CHUNK, _CHUNK)], bufs[j % 2], sin.at[j % 2]
        )
        for j in range(_N_CHUNKS)
    ]
    outs = [
        pltpu.make_async_copy(
            bufs[j % 2], o_hbm.at[pl.ds(base + j * _CHUNK, _CHUNK)], sout.at[j % 2]
        )
        for j in range(_N_CHUNKS)
    ]
    ins[0].start()
    for j in range(_N_CHUNKS):
        if j + 1 < _N_CHUNKS:
            if j - 1 >= 0:
                outs[j - 1].wait()
            ins[j + 1].start()
        ins[j].wait()
        outs[j].start()
    outs[_N_CHUNKS - 2].wait()
    outs[_N_CHUNKS - 1].wait()


def kernel(images):
    b, c, h, w = images.shape
    flat = images.reshape(_TOTAL)
    mesh = plsc.VectorSubcoreMesh(core_axis_name="c", subcore_axis_name="s")
    out = pl.kernel(
        _vsc_body,
        out_type=jax.ShapeDtypeStruct((_TOTAL,), jnp.float32),
        mesh=mesh,
        scratch_types=[
            pltpu.VMEM((_CHUNK,), jnp.float32),
            pltpu.VMEM((_CHUNK,), jnp.float32),
            pltpu.SemaphoreType.DMA((2,)),
            pltpu.SemaphoreType.DMA((2,)),
        ],
    )(flat)
    return out.reshape(b, c, h, w)
